# baseline (speedup 1.0000x reference)
import functools

import jax
import jax.numpy as jnp
from jax.experimental import pallas as pl
from jax.experimental.pallas import tpu as pltpu

NPAD = 128
BETA = 1.0


def _slots_body(x_ref, phi_ref, logits_ref, slots_ref, xb_ref):
    xb = x_ref[0].astype(jnp.bfloat16)
    xb_ref[0] = xb
    lg = jnp.dot(xb, phi_ref[...].astype(jnp.bfloat16),
                 preferred_element_type=jnp.float32)
    logits_ref[0] = lg
    m = jnp.max(lg, axis=0, keepdims=True)
    e = jnp.exp(lg - m)
    disp = (e / jnp.sum(e, axis=0, keepdims=True)).astype(jnp.bfloat16)
    slots = jax.lax.dot_general(disp, xb, (((0,), (0,)), ((), ())),
                                preferred_element_type=jnp.float32)
    slots_ref[0] = slots


def _h_expert_body(xb_ref, w1_ref, b1_ref, se_ref, ew1_ref, eb1_ref,
                   ew2_ref, eb2_ref, h_ref, eouts_ref, *, n_et, nf2):
    c = pl.program_id(0)
    j = pl.program_id(1)
    s = c * pl.num_programs(1) + j

    hb = jnp.dot(xb_ref[...], w1_ref[...].astype(jnp.bfloat16),
                 preferred_element_type=jnp.float32) + b1_ref[...]
    h_ref[...] = jax.nn.gelu(hb).astype(jnp.bfloat16)

    @pl.when(s < n_et)
    def _expert():
        f = s % nf2
        sb = se_ref[0]
        he = jnp.dot(sb, ew1_ref[0].astype(jnp.bfloat16),
                     preferred_element_type=jnp.float32) + eb1_ref[0]
        he = jax.nn.gelu(he).astype(jnp.bfloat16)
        pe = jnp.dot(he, ew2_ref[0].astype(jnp.bfloat16),
                     preferred_element_type=jnp.float32)

        @pl.when(f == 0)
        def _init():
            eouts_ref[...] = jnp.broadcast_to(eb2_ref[...], eouts_ref.shape)

        eouts_ref[...] += pe[None]


def _y_body(h_ref, w2_hbm, b2_ref, lg_ref, outs_ref, y_ref, w2_s, sem,
            *, n_real):
    i = pl.program_id(0)

    @pl.when(i == 0)
    def _fetch_w2():
        cp = pltpu.make_async_copy(w2_hbm, w2_s, sem)
        cp.start()
        cp.wait()

    lg = lg_ref[...]
    lane = jax.lax.broadcasted_iota(jnp.int32, lg.shape, 1)
    lg = jnp.where(lane < n_real, lg, -jnp.inf)
    m = jnp.max(lg, axis=1, keepdims=True)
    e = jnp.exp(lg - m)
    comb = (e / jnp.sum(e, axis=1, keepdims=True)).astype(jnp.bfloat16)
    moe = jnp.dot(comb, outs_ref[0], preferred_element_type=jnp.float32)
    y_ref[...] = (jnp.dot(h_ref[...], w2_s[...],
                          preferred_element_type=jnp.float32)
                  + BETA * moe + b2_ref[...])


def kernel(x, w1, b1, w2, b2, ew1, eb1, ew2, eb2, phi):
    B, S, D = x.shape
    D_FF = w1.shape[1]
    E = ew1.shape[0]
    N = phi.shape[1]
    NS = N // E
    TOKENS = B * S

    phi_p = jnp.pad(phi, ((0, 0), (0, NPAD - N)))

    logits, slots, xb16 = pl.pallas_call(
        _slots_body,
        grid=(B,),
        in_specs=[
            pl.BlockSpec((1, S, D), lambda b: (b, 0, 0)),
            pl.BlockSpec((D, NPAD), lambda b: (0, 0)),
        ],
        out_specs=[
            pl.BlockSpec((1, S, NPAD), lambda b: (b, 0, 0)),
            pl.BlockSpec((1, NPAD, D), lambda b: (b, 0, 0)),
            pl.BlockSpec((1, S, D), lambda b: (b, 0, 0)),
        ],
        out_shape=[
            jax.ShapeDtypeStruct((B, S, NPAD), jnp.float32),
            jax.ShapeDtypeStruct((B, NPAD, D), jnp.float32),
            jax.ShapeDtypeStruct((B, S, D), jnp.bfloat16),
        ],
        compiler_params=pltpu.CompilerParams(
            vmem_limit_bytes=100 * 1024 * 1024),
    )(x, phi_p)

    R = B * NS
    se = (slots[:, :N, :].reshape(B, E, NS, D).transpose(1, 0, 2, 3)
          .reshape(E, R, D).astype(jnp.bfloat16))

    F1 = min(512, D_FF)
    TC = min(2048, TOKENS)
    NJ = D_FF // F1
    nf2 = max(1, D_FF // 512)
    F2 = D_FF // nf2
    n_et = E * nf2
    while (TOKENS // TC) * NJ < n_et:
        TC //= 2

    xf = xb16.reshape(TOKENS, D)

    def _ew_t(c, j):
        return jnp.minimum(c * NJ + j, n_et - 1)

    h, eouts = pl.pallas_call(
        functools.partial(_h_expert_body, n_et=n_et, nf2=nf2),
        grid=(TOKENS // TC, NJ),
        in_specs=[
            pl.BlockSpec((TC, D), lambda c, j: (c, 0)),
            pl.BlockSpec((D, F1), lambda c, j: (0, j)),
            pl.BlockSpec((1, F1), lambda c, j: (0, j)),
            pl.BlockSpec((1, R, D), lambda c, j: (_ew_t(c, j) // nf2, 0, 0)),
            pl.BlockSpec((1, D, F2),
                         lambda c, j: (_ew_t(c, j) // nf2, 0,
                                       _ew_t(c, j) % nf2)),
            pl.BlockSpec((1, 1, F2),
                         lambda c, j: (_ew_t(c, j) // nf2, 0,
                                       _ew_t(c, j) % nf2)),
            pl.BlockSpec((1, F2, D),
                         lambda c, j: (_ew_t(c, j) // nf2,
                                       _ew_t(c, j) % nf2, 0)),
            pl.BlockSpec((1, 1, D), lambda c, j: (_ew_t(c, j) // nf2, 0, 0)),
        ],
        out_specs=[
            pl.BlockSpec((TC, F1), lambda c, j: (c, j)),
            pl.BlockSpec((1, R, D), lambda c, j: (_ew_t(c, j) // nf2, 0, 0)),
        ],
        out_shape=[
            jax.ShapeDtypeStruct((TOKENS, D_FF), jnp.bfloat16),
            jax.ShapeDtypeStruct((E, R, D), jnp.float32),
        ],
    )(xf, w1.astype(jnp.bfloat16), b1.reshape(1, D_FF), se,
      ew1.reshape(E, D, D_FF), eb1.reshape(E, 1, D_FF), ew2,
      eb2.reshape(E, 1, D))

    outs_p = jnp.pad(
        (eouts.reshape(E, B, NS, D).transpose(1, 0, 2, 3).reshape(B, N, D)),
        ((0, 0), (0, NPAD - N), (0, 0))).astype(jnp.bfloat16)

    T = min(256, S)
    tpb = S // T
    y = pl.pallas_call(
        functools.partial(_y_body, n_real=N),
        grid=(TOKENS // T,),
        in_specs=[
            pl.BlockSpec((T, D_FF), lambda i: (i, 0)),
            pl.BlockSpec(memory_space=pl.ANY),
            pl.BlockSpec((1, D), lambda i: (0, 0)),
            pl.BlockSpec((T, NPAD), lambda i: (i, 0)),
            pl.BlockSpec((1, NPAD, D), lambda i: (i // tpb, 0, 0)),
        ],
        out_specs=pl.BlockSpec((T, D), lambda i: (i, 0)),
        out_shape=jax.ShapeDtypeStruct((TOKENS, D), jnp.float32),
        scratch_shapes=[
            pltpu.VMEM((D_FF, D), jnp.bfloat16),
            pltpu.SemaphoreType.DMA,
        ],
    )(h, w2.astype(jnp.bfloat16), b2.reshape(1, D),
      logits.reshape(TOKENS, NPAD), outs_p)

    return y.reshape(B, S, D)
